# baseline (speedup 1.0000x reference)
import functools

import jax
import jax.numpy as jnp
from jax import lax
from jax.experimental import pallas as pl
from jax.experimental.pallas import tpu as pltpu
from jax.experimental.pallas import tpu_sc as plsc

_N = 8192
_IN = 256
_ROWS = 8
_LANES = _N // _ROWS
_K = 10
_L = 16


def _score_kernel(x_ref, w_ref, b_ref, o_ref):
    w = w_ref[...]
    rows = []
    for r in range(_ROWS // 2):
        xc = x_ref[0, pl.ds(r * _LANES, _LANES), :]
        rows.append(
            jax.lax.dot_general(
                w, xc, (((1,), (1,)), ((), ())),
                preferred_element_type=jnp.float32,
            )
        )
    s = jnp.concatenate(rows, axis=1) + b_ref[...]
    o_ref[...] = s.reshape(_N // 2)


def _sc_select_body(scores_hbm, out_hbm, buf, res):
    info = plsc.get_sparse_core_info()
    ns = info.num_subcores
    wid = lax.axis_index("c") * ns + lax.axis_index("s")

    @pl.when(wid < scores_hbm.shape[0] // _N)
    def _work():
        pltpu.sync_copy(scores_hbm.at[pl.ds(wid * _N, _N)], buf)
        ninf = jnp.full((_L,), float("-inf"), jnp.float32)
        pinf = jnp.full((_L,), float("inf"), jnp.float32)
        lane = lax.iota(jnp.int32, _L)

        def step(i, carry):
            t = list(carry[:_K])
            u = list(carry[_K:])
            c = buf[pl.ds(pl.multiple_of(i * _L, _L), _L)]
            d = c
            for j in range(_K):
                t_hi = jnp.maximum(t[j], c)
                c = jnp.minimum(t[j], c)
                t[j] = t_hi
                u_lo = jnp.minimum(u[j], d)
                d = jnp.maximum(u[j], d)
                u[j] = u_lo
            return tuple(t) + tuple(u)

        init = tuple(ninf for _ in range(_K)) + tuple(pinf for _ in range(_K))
        fin = lax.fori_loop(0, _N // _L, step, init)
        t = list(fin[:_K])
        u = list(fin[_K:])

        top_sum = jnp.float32(0.0)
        bot_sum = jnp.float32(0.0)
        for _ in range(_K):
            best_t = jnp.float32(float("-inf"))
            best_u = jnp.float32(float("inf"))
            bl_t = jnp.int32(0)
            bl_u = jnp.int32(0)
            for l in range(_L):
                vt = t[0][l]
                vu = u[0][l]
                bt = vt > best_t
                bu = vu < best_u
                best_t = jnp.where(bt, vt, best_t)
                bl_t = jnp.where(bt, l, bl_t)
                best_u = jnp.where(bu, vu, best_u)
                bl_u = jnp.where(bu, l, bl_u)
            top_sum = top_sum + best_t
            bot_sum = bot_sum + best_u
            hot_t = lane == jnp.full((_L,), bl_t, jnp.int32)
            hot_u = lane == jnp.full((_L,), bl_u, jnp.int32)
            for j in range(_K - 1):
                t[j] = jnp.where(hot_t, t[j + 1], t[j])
                u[j] = jnp.where(hot_u, u[j + 1], u[j])
            t[_K - 1] = jnp.where(hot_t, ninf, t[_K - 1])
            u[_K - 1] = jnp.where(hot_u, pinf, u[_K - 1])

        mean = (top_sum + bot_sum) * jnp.float32(1.0 / (2 * _K))
        mvec = jnp.full((_L,), mean, jnp.float32)
        res[...] = 1.0 / (1.0 + jnp.exp(-mvec))
        pltpu.sync_copy(res, out_hbm.at[wid])


def _score_call(x, W, b2):
    B = x.shape[0]
    x2 = x.reshape(B * 2, _N // 2, _IN)
    return pl.pallas_call(
        _score_kernel,
        grid=(B * 2,),
        in_specs=[
            pl.BlockSpec((1, _N // 2, _IN), lambda i: (i, 0, 0)),
            pl.BlockSpec((1, _IN), lambda i: (0, 0)),
            pl.BlockSpec((1, 1), lambda i: (0, 0)),
        ],
        out_specs=pl.BlockSpec((_N // 2,), lambda i: (i,)),
        out_shape=jax.ShapeDtypeStruct((B * _N,), jnp.float32),
        compiler_params=pltpu.CompilerParams(
            dimension_semantics=("parallel",),
        ),
    )(x2, W, b2)


def _sc_call(scores):
    B = scores.shape[0] // _N
    sc_select = functools.partial(
        pl.kernel,
        mesh=plsc.VectorSubcoreMesh(core_axis_name="c", subcore_axis_name="s"),
        out_type=jax.ShapeDtypeStruct((B, _L), jnp.float32),
        scratch_types=[
            pltpu.VMEM((_N,), jnp.float32),
            pltpu.VMEM((_L,), jnp.float32),
        ],
    )(_sc_select_body)
    return sc_select(scores)[:, 0]


@jax.jit
def kernel(x, W, b):
    scores = _score_call(x, W, jnp.reshape(b, (1, 1)))
    return _sc_call(scores)


# speedup vs baseline: 1.1122x; 1.1122x over previous
import functools

import jax
import jax.numpy as jnp
from jax import lax
from jax.experimental import pallas as pl
from jax.experimental.pallas import tpu as pltpu
from jax.experimental.pallas import tpu_sc as plsc

_N = 8192
_IN = 256
_ROWS = 8
_LANES = _N // _ROWS
_K = 10
_L = 16


def _score_kernel(x_ref, w_ref, b_ref, o_ref):
    w = w_ref[...]
    rows = []
    for r in range(_ROWS):
        xc = x_ref[0, pl.ds(r * _LANES, _LANES), :]
        rows.append(
            jax.lax.dot_general(
                w, xc, (((1,), (1,)), ((), ())),
                preferred_element_type=jnp.float32,
            )
        )
    s = jnp.concatenate(rows, axis=1) + b_ref[...]
    o_ref[...] = s.reshape(_N)


def _sc_select_body(scores_hbm, out_hbm, buf, res):
    info = plsc.get_sparse_core_info()
    ns = info.num_subcores
    wid = lax.axis_index("c") * ns + lax.axis_index("s")

    @pl.when(wid < scores_hbm.shape[0] // _N)
    def _work():
        pltpu.sync_copy(scores_hbm.at[pl.ds(wid * _N, _N)], buf)
        ninf = jnp.full((_L,), float("-inf"), jnp.float32)
        pinf = jnp.full((_L,), float("inf"), jnp.float32)
        lane = lax.iota(jnp.int32, _L)

        def step(i, carry):
            t = list(carry[:_K])
            u = list(carry[_K:])
            c = buf[pl.ds(pl.multiple_of(i * _L, _L), _L)]
            d = c
            for j in range(_K):
                t_hi = jnp.maximum(t[j], c)
                c = jnp.minimum(t[j], c)
                t[j] = t_hi
                u_lo = jnp.minimum(u[j], d)
                d = jnp.maximum(u[j], d)
                u[j] = u_lo
            return tuple(t) + tuple(u)

        init = tuple(ninf for _ in range(_K)) + tuple(pinf for _ in range(_K))
        fin = lax.fori_loop(0, _N // _L, step, init)
        t = list(fin[:_K])
        u = list(fin[_K:])

        top_sum = jnp.float32(0.0)
        bot_sum = jnp.float32(0.0)
        for _ in range(_K):
            best_t = jnp.float32(float("-inf"))
            best_u = jnp.float32(float("inf"))
            bl_t = jnp.int32(0)
            bl_u = jnp.int32(0)
            for l in range(_L):
                vt = t[0][l]
                vu = u[0][l]
                bt = vt > best_t
                bu = vu < best_u
                best_t = jnp.where(bt, vt, best_t)
                bl_t = jnp.where(bt, l, bl_t)
                best_u = jnp.where(bu, vu, best_u)
                bl_u = jnp.where(bu, l, bl_u)
            top_sum = top_sum + best_t
            bot_sum = bot_sum + best_u
            hot_t = lane == jnp.full((_L,), bl_t, jnp.int32)
            hot_u = lane == jnp.full((_L,), bl_u, jnp.int32)
            for j in range(_K - 1):
                t[j] = jnp.where(hot_t, t[j + 1], t[j])
                u[j] = jnp.where(hot_u, u[j + 1], u[j])
            t[_K - 1] = jnp.where(hot_t, ninf, t[_K - 1])
            u[_K - 1] = jnp.where(hot_u, pinf, u[_K - 1])

        mean = (top_sum + bot_sum) * jnp.float32(1.0 / (2 * _K))
        mvec = jnp.full((_L,), mean, jnp.float32)
        res[...] = 1.0 / (1.0 + jnp.exp(-mvec))
        pltpu.sync_copy(res, out_hbm.at[wid])


def _score_call(x, W, b2):
    B = x.shape[0]
    return pl.pallas_call(
        _score_kernel,
        grid=(B,),
        in_specs=[
            pl.BlockSpec((1, _N, _IN), lambda i: (i, 0, 0)),
            pl.BlockSpec((1, _IN), lambda i: (0, 0)),
            pl.BlockSpec((1, 1), lambda i: (0, 0)),
        ],
        out_specs=pl.BlockSpec((_N,), lambda i: (i,)),
        out_shape=jax.ShapeDtypeStruct((B * _N,), jnp.float32),
        compiler_params=pltpu.CompilerParams(
            dimension_semantics=("parallel",),
        ),
    )(x, W, b2)


def _sc_call(scores):
    B = scores.shape[0] // _N
    sc_select = functools.partial(
        pl.kernel,
        mesh=plsc.VectorSubcoreMesh(core_axis_name="c", subcore_axis_name="s"),
        out_type=jax.ShapeDtypeStruct((B, _L), jnp.float32),
        scratch_types=[
            pltpu.VMEM((_N,), jnp.float32),
            pltpu.VMEM((_L,), jnp.float32),
        ],
    )(_sc_select_body)
    return sc_select(scores)[:, 0]


@jax.jit
def kernel(x, W, b):
    scores = _score_call(x, W, jnp.reshape(b, (1, 1)))
    return _sc_call(scores)
